# baseline (speedup 1.0000x reference)
import jax
import jax.numpy as jnp
from jax.experimental import pallas as pl


def _add_pe_kernel(x_ref, pe_ref, o_ref):
    o_ref[...] = x_ref[...] + pe_ref[...]


def kernel(x, pe):
    batch, seq_len, dim = x.shape
    dblk = 512
    grid = (dim // dblk, batch)
    return pl.pallas_call(
        _add_pe_kernel,
        grid=grid,
        in_specs=[
            pl.BlockSpec((1, seq_len, dblk), lambda d, b: (b, 0, d)),
            pl.BlockSpec((seq_len, dblk), lambda d, b: (0, d)),
        ],
        out_specs=pl.BlockSpec((1, seq_len, dblk), lambda d, b: (b, 0, d)),
        out_shape=jax.ShapeDtypeStruct(x.shape, x.dtype),
    )(x, pe)


# speedup vs baseline: 1.0076x; 1.0076x over previous
import jax
import jax.numpy as jnp
from jax.experimental import pallas as pl


def _add_pe_kernel(x_ref, pe_ref, o_ref):
    o_ref[...] = x_ref[...] + pe_ref[...]


def kernel(x, pe):
    batch, seq_len, dim = x.shape
    sblk = 1024
    grid = (seq_len // sblk, batch)
    return pl.pallas_call(
        _add_pe_kernel,
        grid=grid,
        in_specs=[
            pl.BlockSpec((1, sblk, dim), lambda s, b: (b, s, 0)),
            pl.BlockSpec((sblk, dim), lambda s, b: (s, 0)),
        ],
        out_specs=pl.BlockSpec((1, sblk, dim), lambda s, b: (b, s, 0)),
        out_shape=jax.ShapeDtypeStruct(x.shape, x.dtype),
    )(x, pe)
